# baseline (speedup 1.0000x reference)
import functools

import jax
import jax.numpy as jnp
from jax.experimental import pallas as pl
from jax.experimental.pallas import tpu as pltpu

E = 8
D = 768
F = 768
TBLK = 256


def _moe_body(h_ref, wr_ref, rb_ref, we_ref, eb_ref, out_ref):
    h = h_ref[...].astype(jnp.bfloat16)
    logits = jnp.dot(h, wr_ref[...].astype(jnp.bfloat16),
                     preferred_element_type=jnp.float32)
    logits = logits + rb_ref[...]
    chunks = [logits[:, e * F:(e + 1) * F] for e in range(E)]

    m1 = jnp.maximum(chunks[0], chunks[1])
    m2 = jnp.minimum(chunks[0], chunks[1])
    for e in range(2, E):
        x = chunks[e]
        m2 = jnp.maximum(m2, jnp.minimum(m1, x))
        m1 = jnp.maximum(m1, x)

    t = jnp.exp(m2 - m1)
    w1 = 1.0 / (1.0 + t)
    w2 = 1.0 - w1

    wt_cols = []
    for e in range(E):
        x = chunks[e]
        mass = jnp.where(x == m1, w1, jnp.where(x == m2, w2, 0.0))
        wt_cols.append(jnp.sum(mass, axis=1, keepdims=True))

    wt = jnp.concatenate(wt_cols, axis=1)
    acc = jnp.dot(wt, eb_ref[...], preferred_element_type=jnp.float32)
    for e in range(E):
        pe = jnp.dot(h, we_ref[e].astype(jnp.bfloat16),
                     preferred_element_type=jnp.float32)
        acc = acc + pe * wt_cols[e]
    out_ref[...] = acc


@jax.jit
def _run(h, wr, rb, we, eb):
    T = h.shape[0]
    grid = (T // TBLK,)
    return pl.pallas_call(
        _moe_body,
        grid=grid,
        in_specs=[
            pl.BlockSpec((TBLK, D), lambda i: (i, 0)),
            pl.BlockSpec((D, E * F), lambda i: (0, 0)),
            pl.BlockSpec((1, E * F), lambda i: (0, 0)),
            pl.BlockSpec((E, D, F), lambda i: (0, 0, 0)),
            pl.BlockSpec((E, F), lambda i: (0, 0)),
        ],
        out_specs=pl.BlockSpec((TBLK, F), lambda i: (i, 0)),
        out_shape=jax.ShapeDtypeStruct((T, F), jnp.float32),
        compiler_params=pltpu.CompilerParams(
            dimension_semantics=("parallel",),
        ),
    )(h, wr, rb, we, eb)


def kernel(hidden_states, router_W, router_b, expert_W, expert_b):
    B, S, Dh = hidden_states.shape
    T = B * S
    h = hidden_states.reshape(T, Dh)
    wr = router_W.reshape(Dh, F, E).transpose(0, 2, 1).reshape(Dh, E * F)
    rb = router_b.reshape(F, E).T.reshape(1, E * F).astype(jnp.float32)
    we = expert_W
    eb = expert_b.astype(jnp.float32)
    out = _run(h, wr, rb, we, eb)
    return out.reshape(B, S, F)


# speedup vs baseline: 2.0761x; 2.0761x over previous
import numpy as np

import jax
import jax.numpy as jnp
from jax.experimental import pallas as pl
from jax.experimental.pallas import tpu as pltpu

E = 8
D = 768
F = 768
TBLK = 256
BAND = 1024
NBANDS = (F * E) // BAND
DSLAB = 192
NPRO = D // DSLAB

_pnp = np.zeros((BAND, BAND), dtype=np.float32)
for _u in range(BAND // E):
    for _e in range(E):
        _pnp[E * _u + _e, 128 * _e + _u] = 1.0


def _chunk(planar, e):
    return jnp.concatenate(
        [planar[:, BAND * s + 128 * e:BAND * s + 128 * e + 128]
         for s in range(NBANDS)], axis=1)


def _moe_body(h_ref, wr_ref, rb_ref, we_ref, eb_ref, p_ref, out_ref,
              wrp_ref, rbp_ref):
    i = pl.program_id(0)

    @pl.when(i < NPRO)
    def _deinterleave():
        wr_bf = wr_ref[...].astype(jnp.bfloat16)
        p = p_ref[...]
        for s in range(NBANDS):
            band = jnp.dot(wr_bf[:, s * BAND:(s + 1) * BAND], p,
                           preferred_element_type=jnp.float32)
            wrp_ref[pl.ds(i * DSLAB, DSLAB),
                    s * BAND:(s + 1) * BAND] = band.astype(jnp.bfloat16)

    @pl.when(i == 0)
    def _bias():
        rb_bf = rb_ref[...].astype(jnp.bfloat16)
        p = p_ref[...]
        for s in range(NBANDS):
            rbp_ref[:, s * BAND:(s + 1) * BAND] = jnp.dot(
                rb_bf[:, s * BAND:(s + 1) * BAND], p,
                preferred_element_type=jnp.float32)

    @pl.when(i >= NPRO)
    def _tile():
        h = h_ref[...].astype(jnp.bfloat16)
        logits = jnp.dot(h, wrp_ref[...], preferred_element_type=jnp.float32)
        logits = logits + rbp_ref[...]
        chunks = [_chunk(logits, e) for e in range(E)]

        m1 = jnp.maximum(chunks[0], chunks[1])
        m2 = jnp.minimum(chunks[0], chunks[1])
        for e in range(2, E):
            x = chunks[e]
            m2 = jnp.maximum(m2, jnp.minimum(m1, x))
            m1 = jnp.maximum(m1, x)

        t = jnp.exp(m2 - m1)
        w1 = 1.0 / (1.0 + t)
        w2 = 1.0 - w1

        wt_cols = []
        for e in range(E):
            x = chunks[e]
            mass = jnp.where(x == m1, w1, jnp.where(x == m2, w2, 0.0))
            wt_cols.append(jnp.sum(mass, axis=1, keepdims=True))

        wt = jnp.concatenate(wt_cols, axis=1)
        acc = jnp.dot(wt, eb_ref[...], preferred_element_type=jnp.float32)
        for e in range(E):
            pe = jnp.dot(h, we_ref[e].astype(jnp.bfloat16),
                         preferred_element_type=jnp.float32)
            acc = acc + pe * wt_cols[e]
        out_ref[...] = acc


@jax.jit
def _run(h, wr, rb, we, eb):
    T = h.shape[0]
    ntiles = T // TBLK
    grid = (NPRO + ntiles,)

    def tile_idx(i):
        return jnp.maximum(i - NPRO, 0)

    return pl.pallas_call(
        _moe_body,
        grid=grid,
        in_specs=[
            pl.BlockSpec((TBLK, D), lambda i: (tile_idx(i), 0)),
            pl.BlockSpec((DSLAB, F * E), lambda i: (jnp.minimum(i, NPRO - 1), 0)),
            pl.BlockSpec((1, F * E), lambda i: (0, 0)),
            pl.BlockSpec((E, D, F), lambda i: (0, 0, 0)),
            pl.BlockSpec((E, F), lambda i: (0, 0)),
            pl.BlockSpec((BAND, BAND), lambda i: (0, 0)),
        ],
        out_specs=pl.BlockSpec((TBLK, F), lambda i: (tile_idx(i), 0)),
        out_shape=jax.ShapeDtypeStruct((T, F), jnp.float32),
        scratch_shapes=[
            pltpu.VMEM((D, F * E), jnp.bfloat16),
            pltpu.VMEM((1, F * E), jnp.float32),
        ],
    )(h, wr, rb, we, eb, jnp.asarray(_pnp, dtype=jnp.bfloat16))


def kernel(hidden_states, router_W, router_b, expert_W, expert_b):
    B, S, Dh = hidden_states.shape
    T = B * S
    h = hidden_states.reshape(T, Dh)
    rb = router_b.reshape(1, E * F)
    out = _run(h, router_W, rb, expert_W, expert_b)
    return out.reshape(B, S, F)


# speedup vs baseline: 2.1662x; 1.0434x over previous
import numpy as np

import jax
import jax.numpy as jnp
from jax.experimental import pallas as pl
from jax.experimental.pallas import tpu as pltpu

E = 8
D = 768
F = 768
TBLK = 256
BAND = 1024
NBANDS = (F * E) // BAND
DSLAB = 192
NPRO = D // DSLAB

_pnp = np.zeros((BAND, BAND), dtype=np.float32)
for _u in range(BAND // E):
    for _e in range(E):
        _pnp[E * _u + _e, 128 * _e + _u] = 1.0


def _chunk(planar, e):
    return jnp.concatenate(
        [planar[:, BAND * s + 128 * e:BAND * s + 128 * e + 128]
         for s in range(NBANDS)], axis=1)


def _moe_body(h_ref, wr_ref, rb_ref, we_ref, eb_ref, p_ref, out_ref,
              wrp_ref, rbp_ref, webf_ref):
    i = pl.program_id(0)

    @pl.when(i < NPRO)
    def _deinterleave():
        wr_bf = wr_ref[...].astype(jnp.bfloat16)
        p = p_ref[...]
        for s in range(NBANDS):
            band = jnp.dot(wr_bf[:, s * BAND:(s + 1) * BAND], p,
                           preferred_element_type=jnp.float32)
            wrp_ref[pl.ds(i * DSLAB, DSLAB),
                    s * BAND:(s + 1) * BAND] = band.astype(jnp.bfloat16)
        webf_ref[pl.ds(2 * i, 2)] = we_ref[...].astype(jnp.bfloat16)

    @pl.when(i == 0)
    def _bias():
        rb_bf = rb_ref[...].astype(jnp.bfloat16)
        p = p_ref[...]
        for s in range(NBANDS):
            rbp_ref[:, s * BAND:(s + 1) * BAND] = jnp.dot(
                rb_bf[:, s * BAND:(s + 1) * BAND], p,
                preferred_element_type=jnp.float32)

    @pl.when(i >= NPRO)
    def _tile():
        h = h_ref[...].astype(jnp.bfloat16)
        logits = jnp.dot(h, wrp_ref[...], preferred_element_type=jnp.float32)
        logits = logits + rbp_ref[...]
        chunks = [_chunk(logits, e) for e in range(E)]

        m1 = jnp.maximum(chunks[0], chunks[1])
        m2 = jnp.minimum(chunks[0], chunks[1])
        for e in range(2, E):
            x = chunks[e]
            m2 = jnp.maximum(m2, jnp.minimum(m1, x))
            m1 = jnp.maximum(m1, x)

        t = jnp.exp(m2 - m1)
        w1 = 1.0 / (1.0 + t)
        w2 = 1.0 - w1

        wt_cols = []
        for e in range(E):
            x = chunks[e]
            mass = jnp.where(x == m1, w1, jnp.where(x == m2, w2, 0.0))
            wt_cols.append(jnp.sum(mass, axis=1, keepdims=True))

        wt = jnp.concatenate(wt_cols, axis=1)
        acc = jnp.dot(wt, eb_ref[...], preferred_element_type=jnp.float32)
        for e in range(E):
            pe = jnp.dot(h, webf_ref[e], preferred_element_type=jnp.float32)
            acc = acc + pe * wt_cols[e]
        out_ref[...] = acc


@jax.jit
def _run(h, wr, rb, we, eb):
    T = h.shape[0]
    ntiles = T // TBLK
    grid = (NPRO + ntiles,)

    def tile_idx(i):
        return jnp.maximum(i - NPRO, 0)

    return pl.pallas_call(
        _moe_body,
        grid=grid,
        in_specs=[
            pl.BlockSpec((TBLK, D), lambda i: (tile_idx(i), 0)),
            pl.BlockSpec((DSLAB, F * E), lambda i: (jnp.minimum(i, NPRO - 1), 0)),
            pl.BlockSpec((1, F * E), lambda i: (0, 0)),
            pl.BlockSpec((2, D, F), lambda i: (jnp.minimum(i, NPRO - 1), 0, 0)),
            pl.BlockSpec((E, F), lambda i: (0, 0)),
            pl.BlockSpec((BAND, BAND), lambda i: (0, 0)),
        ],
        out_specs=pl.BlockSpec((TBLK, F), lambda i: (tile_idx(i), 0)),
        out_shape=jax.ShapeDtypeStruct((T, F), jnp.float32),
        scratch_shapes=[
            pltpu.VMEM((D, F * E), jnp.bfloat16),
            pltpu.VMEM((1, F * E), jnp.float32),
            pltpu.VMEM((E, D, F), jnp.bfloat16),
        ],
    )(h, wr, rb, we, eb, jnp.asarray(_pnp, dtype=jnp.bfloat16))


def kernel(hidden_states, router_W, router_b, expert_W, expert_b):
    B, S, Dh = hidden_states.shape
    T = B * S
    h = hidden_states.reshape(T, Dh)
    rb = router_b.reshape(1, E * F)
    out = _run(h, router_W, rb, expert_W, expert_b)
    return out.reshape(B, S, F)


# speedup vs baseline: 2.1884x; 1.0102x over previous
import numpy as np

import jax
import jax.numpy as jnp
from jax.experimental import pallas as pl
from jax.experimental.pallas import tpu as pltpu

E = 8
D = 768
F = 768
TBLK = 256
BAND = 1024
NBANDS = (F * E) // BAND
DSLAB = 192
NPRO = D // DSLAB

_pnp = np.zeros((BAND, BAND), dtype=np.float32)
for _u in range(BAND // E):
    for _e in range(E):
        _pnp[E * _u + _e, 128 * _e + _u] = 1.0


def _chunk(planar, e):
    return jnp.concatenate(
        [planar[:, BAND * s + 128 * e:BAND * s + 128 * e + 128]
         for s in range(NBANDS)], axis=1)


def _moe_body(h_ref, wr_ref, rb_ref, we_ref, eb_ref, p_ref, out_ref,
              wrp_ref, rbp_ref, webf_ref):
    i = pl.program_id(0)

    @pl.when(i < NPRO)
    def _deinterleave():
        wr_bf = wr_ref[...].astype(jnp.bfloat16)
        p = p_ref[...]
        for s in range(NBANDS):
            band = jnp.dot(wr_bf[:, s * BAND:(s + 1) * BAND], p,
                           preferred_element_type=jnp.float32)
            wrp_ref[pl.ds(i * DSLAB, DSLAB),
                    s * BAND:(s + 1) * BAND] = band.astype(jnp.bfloat16)
        webf_ref[pl.ds(2 * i, 2)] = we_ref[...].astype(jnp.bfloat16)

    @pl.when(i == 0)
    def _bias():
        rb_bf = rb_ref[...].astype(jnp.bfloat16)
        p = p_ref[...]
        for s in range(NBANDS):
            rbp_ref[:, s * BAND:(s + 1) * BAND] = jnp.dot(
                rb_bf[:, s * BAND:(s + 1) * BAND], p,
                preferred_element_type=jnp.float32).astype(jnp.bfloat16)

    @pl.when(i >= NPRO)
    def _tile():
        h = h_ref[...].astype(jnp.bfloat16)
        logits = jnp.dot(h, wrp_ref[...],
                         preferred_element_type=jnp.float32
                         ).astype(jnp.bfloat16)
        logits = logits + rbp_ref[...]
        chunks = [_chunk(logits, e) for e in range(E)]

        m1 = jnp.maximum(chunks[0], chunks[1])
        m2 = jnp.minimum(chunks[0], chunks[1])
        for e in range(2, E):
            x = chunks[e]
            m2 = jnp.maximum(m2, jnp.minimum(m1, x))
            m1 = jnp.maximum(m1, x)

        t = jnp.exp((m2 - m1).astype(jnp.float32))
        w1 = (1.0 / (1.0 + t)).astype(jnp.bfloat16)
        w2 = (jnp.bfloat16(1.0) - w1).astype(jnp.bfloat16)

        wt_cols = []
        zero = jnp.zeros_like(w1)
        for e in range(E):
            x = chunks[e]
            mass = jnp.where(x == m1, w1, jnp.where(x == m2, w2, zero))
            wt_cols.append(jnp.sum(mass.astype(jnp.float32), axis=1,
                                   keepdims=True))

        wt = jnp.concatenate(wt_cols, axis=1)
        acc = jnp.dot(wt, eb_ref[...], preferred_element_type=jnp.float32)
        for e in range(E):
            pe = jnp.dot(h, webf_ref[e], preferred_element_type=jnp.float32)
            acc = acc + pe * wt_cols[e]
        out_ref[...] = acc


@jax.jit
def _run(h, wr, rb, we, eb):
    T = h.shape[0]
    ntiles = T // TBLK
    grid = (NPRO + ntiles,)

    def tile_idx(i):
        return jnp.maximum(i - NPRO, 0)

    return pl.pallas_call(
        _moe_body,
        grid=grid,
        in_specs=[
            pl.BlockSpec((TBLK, D), lambda i: (tile_idx(i), 0)),
            pl.BlockSpec((DSLAB, F * E), lambda i: (jnp.minimum(i, NPRO - 1), 0)),
            pl.BlockSpec((1, F * E), lambda i: (0, 0)),
            pl.BlockSpec((2, D, F), lambda i: (jnp.minimum(i, NPRO - 1), 0, 0)),
            pl.BlockSpec((E, F), lambda i: (0, 0)),
            pl.BlockSpec((BAND, BAND), lambda i: (0, 0)),
        ],
        out_specs=pl.BlockSpec((TBLK, F), lambda i: (tile_idx(i), 0)),
        out_shape=jax.ShapeDtypeStruct((T, F), jnp.float32),
        scratch_shapes=[
            pltpu.VMEM((D, F * E), jnp.bfloat16),
            pltpu.VMEM((1, F * E), jnp.bfloat16),
            pltpu.VMEM((E, D, F), jnp.bfloat16),
        ],
    )(h, wr, rb, we, eb, jnp.asarray(_pnp, dtype=jnp.bfloat16))


def kernel(hidden_states, router_W, router_b, expert_W, expert_b):
    B, S, Dh = hidden_states.shape
    T = B * S
    h = hidden_states.reshape(T, Dh)
    rb = router_b.reshape(1, E * F)
    out = _run(h, router_W, rb, expert_W, expert_b)
    return out.reshape(B, S, F)


# speedup vs baseline: 2.2879x; 1.0455x over previous
import numpy as np

import jax
import jax.numpy as jnp
from jax.experimental import pallas as pl
from jax.experimental.pallas import tpu as pltpu

E = 8
D = 768
F = 768
TBLK = 512
BAND = 1024
NBANDS = (F * E) // BAND
DSLAB = 192
NPRO = D // DSLAB

_pnp = np.zeros((BAND, BAND), dtype=np.float32)
for _u in range(BAND // E):
    for _e in range(E):
        _pnp[E * _u + _e, 128 * _e + _u] = 1.0


def _chunk(planar, e):
    return jnp.concatenate(
        [planar[:, BAND * s + 128 * e:BAND * s + 128 * e + 128]
         for s in range(NBANDS)], axis=1)


def _moe_body(h_ref, wr_ref, rb_ref, we_ref, eb_ref, p_ref, out_ref,
              wrp_ref, rbp_ref, webf_ref):
    i = pl.program_id(0)

    @pl.when(i < NPRO)
    def _deinterleave():
        wr_bf = wr_ref[...].astype(jnp.bfloat16)
        p = p_ref[...]
        for s in range(NBANDS):
            band = jnp.dot(wr_bf[:, s * BAND:(s + 1) * BAND], p,
                           preferred_element_type=jnp.float32)
            wrp_ref[pl.ds(i * DSLAB, DSLAB),
                    s * BAND:(s + 1) * BAND] = band.astype(jnp.bfloat16)
        webf_ref[pl.ds(2 * i, 2)] = we_ref[...].astype(jnp.bfloat16)

    @pl.when(i == 0)
    def _bias():
        rb_bf = rb_ref[...].astype(jnp.bfloat16)
        p = p_ref[...]
        for s in range(NBANDS):
            rbp_ref[:, s * BAND:(s + 1) * BAND] = jnp.dot(
                rb_bf[:, s * BAND:(s + 1) * BAND], p,
                preferred_element_type=jnp.float32).astype(jnp.bfloat16)

    @pl.when(i >= NPRO)
    def _tile():
        h = h_ref[...].astype(jnp.bfloat16)
        logits = jnp.dot(h, wrp_ref[...],
                         preferred_element_type=jnp.float32
                         ).astype(jnp.bfloat16)
        logits = logits + rbp_ref[...]
        chunks = [_chunk(logits, e) for e in range(E)]

        m1 = jnp.maximum(chunks[0], chunks[1])
        m2 = jnp.minimum(chunks[0], chunks[1])
        for e in range(2, E):
            x = chunks[e]
            m2 = jnp.maximum(m2, jnp.minimum(m1, x))
            m1 = jnp.maximum(m1, x)

        t = jnp.exp((m2 - m1).astype(jnp.float32))
        w1 = (1.0 / (1.0 + t)).astype(jnp.bfloat16)
        w2 = (jnp.bfloat16(1.0) - w1).astype(jnp.bfloat16)

        wt_cols = []
        zero = jnp.zeros_like(w1)
        for e in range(E):
            x = chunks[e]
            mass = jnp.where(x == m1, w1, jnp.where(x == m2, w2, zero))
            wt_cols.append(jnp.sum(mass.astype(jnp.float32), axis=1,
                                   keepdims=True))

        wt = jnp.concatenate(wt_cols, axis=1)
        acc = jnp.dot(wt, eb_ref[...], preferred_element_type=jnp.float32)
        for e in range(E):
            pe = jnp.dot(h, webf_ref[e], preferred_element_type=jnp.float32)
            acc = acc + pe * wt_cols[e]
        out_ref[...] = acc


@jax.jit
def _run(h, wr, rb, we, eb):
    T = h.shape[0]
    ntiles = T // TBLK
    grid = (NPRO + ntiles,)

    def tile_idx(i):
        return jnp.maximum(i - NPRO, 0)

    return pl.pallas_call(
        _moe_body,
        grid=grid,
        in_specs=[
            pl.BlockSpec((TBLK, D), lambda i: (tile_idx(i), 0)),
            pl.BlockSpec((DSLAB, F * E), lambda i: (jnp.minimum(i, NPRO - 1), 0)),
            pl.BlockSpec((1, F * E), lambda i: (0, 0)),
            pl.BlockSpec((2, D, F), lambda i: (jnp.minimum(i, NPRO - 1), 0, 0)),
            pl.BlockSpec((E, F), lambda i: (0, 0)),
            pl.BlockSpec((BAND, BAND), lambda i: (0, 0)),
        ],
        out_specs=pl.BlockSpec((TBLK, F), lambda i: (tile_idx(i), 0)),
        out_shape=jax.ShapeDtypeStruct((T, F), jnp.float32),
        scratch_shapes=[
            pltpu.VMEM((D, F * E), jnp.bfloat16),
            pltpu.VMEM((1, F * E), jnp.bfloat16),
            pltpu.VMEM((E, D, F), jnp.bfloat16),
        ],
    )(h, wr, rb, we, eb, jnp.asarray(_pnp, dtype=jnp.bfloat16))


def kernel(hidden_states, router_W, router_b, expert_W, expert_b):
    B, S, Dh = hidden_states.shape
    T = B * S
    h = hidden_states.reshape(T, Dh)
    rb = router_b.reshape(1, E * F)
    out = _run(h, router_W, rb, expert_W, expert_b)
    return out.reshape(B, S, F)
